# baseline (speedup 1.0000x reference)
import functools

import jax
import jax.numpy as jnp
from jax import lax
from jax.experimental import pallas as pl
from jax.experimental.pallas import tpu as pltpu
from jax.experimental.pallas import tpu_sc as plsc

N_NODES = 10000
E_EDGES = 320000
N_CH = 4

NC = 2
NS = 16
NW = NC * NS
B = 80
EPT = E_EDGES // NW
NB = EPT // B
NBUF = 5
NPT = N_NODES // NS


def _proj2_body(a_ref, b_ref, wa_ref, wb_ref, o_ref):
    o_ref[:] = (jnp.dot(a_ref[:], wa_ref[:], preferred_element_type=jnp.float32)
                + jnp.dot(b_ref[:], wb_ref[:], preferred_element_type=jnp.float32))


def _mid_body(d, p_ref, b_ref, w_ref, o_ref):
    h = jax.nn.relu(p_ref[0] + p_ref[1] + b_ref[:])[:, :d]
    o_ref[:] = jnp.dot(h, w_ref[:], preferred_element_type=jnp.float32)


def _final_body(d, p_ref, b3_ref, l1w_ref, l1b_ref, l2w_ref, l2b_ref,
                l3w_ref, l3b_ref, o_ref):
    h = jax.nn.relu(p_ref[0] + p_ref[1] + b3_ref[:])[:, :d]
    h = jax.nn.relu(jnp.dot(h, l1w_ref[:].T, preferred_element_type=jnp.float32) + l1b_ref[:])
    h = jax.nn.relu(jnp.dot(h, l2w_ref[:].T, preferred_element_type=jnp.float32) + l2b_ref[:])
    z = jnp.sum(h * l3w_ref[:], axis=1, keepdims=True) + l3b_ref[0]
    o_ref[:] = jax.nn.sigmoid(z)


def _tc_call(body, out_shape, *args):
    return pl.pallas_call(
        body,
        out_shape=jax.ShapeDtypeStruct(out_shape, jnp.float32),
    )(*args)


def _vals_t_body(x_ref, o_ref):
    o_ref[:] = x_ref[:].T


def _vals_transpose(vals):
    blk = 3200
    return pl.pallas_call(
        _vals_t_body,
        grid=(E_EDGES // blk,),
        in_specs=[pl.BlockSpec((blk, N_CH), lambda w: (w, 0))],
        out_specs=pl.BlockSpec((N_CH, blk), lambda w: (0, w)),
        out_shape=jax.ShapeDtypeStruct((N_CH, E_EDGES), jnp.float32),
    )(vals)


def _sc_edge_kernel(dp):
    R = N_CH * dp
    nv = dp // 16

    def body(y_hbm, src_hbm, dst_hbm, vals_hbm, out_hbm,
             src_v, dst_v, vals_v, rows_v, msgs_v, zbuf_v, acc_sh,
             g0, g1, g2, g3, g4, v0, v1, v2, v3, v4, s0, s1, s2, s3, s4):
        gsems = (g0, g1, g2, g3, g4)
        vsems = (v0, v1, v2, v3, v4)
        ssems = (s0, s1, s2, s3, s4)
        cid = lax.axis_index("c")
        sid = lax.axis_index("s")
        wid = cid * NS + sid
        ebase = wid * EPT

        def zrow(r, carry):
            for k in range(nv):
                zbuf_v[r, pl.ds(16 * k, 16)] = jnp.zeros((16,), jnp.float32)
            return carry
        lax.fori_loop(0, NPT // 5, zrow, 0)
        for j in range(5):
            pltpu.sync_copy(zbuf_v,
                            acc_sh.at[pl.ds(sid * NPT + j * (NPT // 5), NPT // 5)])

        pltpu.sync_copy(src_hbm.at[pl.ds(ebase, EPT)], src_v)
        plsc.subcore_barrier()

        def start_rows(i, b):
            pltpu.async_copy(y_hbm.at[src_v.at[pl.ds(i * B, B)]],
                             rows_v.at[b], gsems[b])

        def wait_rows(i, b):
            pltpu.make_async_copy(y_hbm.at[src_v.at[pl.ds(i * B, B)]],
                                  rows_v.at[b], gsems[b]).wait()

        def start_dstvals(i, b):
            pltpu.async_copy(dst_hbm.at[pl.ds(ebase + i * B, B)],
                             dst_v.at[b], vsems[b])
            pltpu.async_copy(vals_hbm.at[:, pl.ds(ebase + i * B, B)],
                             vals_v.at[b], vsems[b])

        def wait_dstvals(i, b):
            pltpu.make_async_copy(dst_hbm.at[pl.ds(ebase + i * B, B)],
                                  dst_v.at[b], vsems[b]).wait()
            pltpu.make_async_copy(vals_hbm.at[:, pl.ds(ebase + i * B, B)],
                                  vals_v.at[b], vsems[b]).wait()

        def start_scatter(b):
            pltpu.async_copy(msgs_v.at[b], acc_sh.at[dst_v.at[b]], ssems[b],
                             add=True)

        def wait_scatter(b):
            pltpu.make_async_copy(msgs_v.at[b], acc_sh.at[dst_v.at[b]],
                                  ssems[b]).wait()

        def compute(b):
            def block16(t, c2):
                vcs = [vals_v[b, c, pl.ds(t * 16, 16)] for c in range(N_CH)]
                for le in range(16):
                    e = t * 16 + le
                    accs = [None] * nv
                    for c in range(N_CH):
                        s = vcs[c].at[jnp.full((16,), le, jnp.int32)].get(
                            mode="promise_in_bounds")
                        for k in range(nv):
                            w = s * rows_v[b, e, pl.ds(c * dp + 16 * k, 16)]
                            accs[k] = w if accs[k] is None else accs[k] + w
                    for k in range(nv):
                        msgs_v[b, e, pl.ds(16 * k, 16)] = accs[k]
                return c2
            lax.fori_loop(0, B // 16, block16, 0)

        start_dstvals(0, 0)
        for i in range(NBUF - 1):
            start_rows(i, i)

        def quint(q, carry):
            p0 = q * NBUF
            for b in range(NBUF):
                p = p0 + b

                @pl.when(p >= NBUF - 1)
                def _():
                    wait_scatter((b + 1) % NBUF)

                @pl.when(p + 1 < NB)
                def _():
                    start_dstvals(p + 1, (b + 1) % NBUF)

                @pl.when(p + NBUF - 1 < NB)
                def _():
                    start_rows(p + NBUF - 1, (b + NBUF - 1) % NBUF)
                wait_rows(p, b)
                wait_dstvals(p, b)
                compute(b)
                start_scatter(b)
            return carry
        lax.fori_loop(0, NB // NBUF, quint, 0)
        for b in range((NB - NBUF + 1) % NBUF, NBUF):
            wait_scatter(b)

        plsc.subcore_barrier()

        @pl.when(sid == 0)
        def _():
            pltpu.sync_copy(acc_sh, out_hbm.at[cid])

    mesh = plsc.VectorSubcoreMesh(core_axis_name="c", subcore_axis_name="s",
                                  num_cores=NC, num_subcores=NS)
    return pl.kernel(
        body,
        out_type=jax.ShapeDtypeStruct((NC, N_NODES, dp), jnp.float32),
        mesh=mesh,
        compiler_params=pltpu.CompilerParams(use_tc_tiling_on_sc=False),
        scratch_types=[
            pltpu.VMEM((EPT,), jnp.int32),
            pltpu.VMEM((NBUF, B), jnp.int32),
            pltpu.VMEM((NBUF, N_CH, B), jnp.float32),
            pltpu.VMEM((NBUF, B, R), jnp.float32),
            pltpu.VMEM((NBUF, B, dp), jnp.float32),
            pltpu.VMEM((NPT // 5, dp), jnp.float32),
            pltpu.VMEM_SHARED((N_NODES, dp), jnp.float32),
        ] + [pltpu.SemaphoreType.DMA] * 15,
    )


def _stack_pad(W, dp):
    d_in, d = W.shape[1], W.shape[2]
    Wt = jnp.transpose(W, (1, 0, 2))
    Wt = jnp.pad(Wt, ((0, 0), (0, 0), (0, dp - d)))
    return Wt.reshape(d_in, N_CH * dp)


def kernel(one_hot, features, gemme_features, a_res_indices, a_res_values,
           W1, b1, W2, b2, W3, b3, l1w, l1b, l2w, l2b, l3w, l3b):
    src = a_res_indices[0].astype(jnp.int32)
    dst = a_res_indices[1].astype(jnp.int32)
    vals_t = _vals_transpose(a_res_values)

    d1, d2, d3 = W1.shape[2], W2.shape[2], W3.shape[2]
    dp1, dp2, dp3 = 32, 16, 16
    W1s = _stack_pad(W1, dp1)
    W2s = _stack_pad(W2, dp2)
    W3s = _stack_pad(W3, dp3)
    b1p = jnp.pad(b1, (0, dp1 - d1))
    b2p = jnp.pad(b2, (0, dp2 - d2))
    b3p = jnp.pad(b3, (0, dp3 - d3))

    edge1 = _sc_edge_kernel(dp1)
    edge23 = _sc_edge_kernel(dp2)

    r = one_hot.shape[1]
    Y1 = _tc_call(_proj2_body, (N_NODES, N_CH * dp1),
                  one_hot, features, W1s[:r], W1s[r:])
    p1 = edge1(Y1, src, dst, vals_t)

    Y2 = _tc_call(functools.partial(_mid_body, d1), (N_NODES, N_CH * dp2),
                  p1, b1p, W2s)
    p2 = edge23(Y2, src, dst, vals_t)

    Y3 = _tc_call(functools.partial(_mid_body, d2), (N_NODES, N_CH * dp3),
                  p2, b2p, W3s)
    p3 = edge23(Y3, src, dst, vals_t)

    return _tc_call(functools.partial(_final_body, d3), (N_NODES, 1),
                    p3, b3p, l1w, l1b, l2w, l2b, l3w, l3b)


# speedup vs baseline: 1.0796x; 1.0796x over previous
import functools

import jax
import jax.numpy as jnp
from jax import lax
from jax.experimental import pallas as pl
from jax.experimental.pallas import tpu as pltpu
from jax.experimental.pallas import tpu_sc as plsc

N_NODES = 10000
E_EDGES = 320000
N_CH = 4

NC = 2
NS = 16
NW = NC * NS
B = 80
EPT = E_EDGES // NW
NB = EPT // B
NBUF = 5
NPT = N_NODES // NS


def _proj2_body(a_ref, b_ref, wa_ref, wb_ref, o_ref):
    o_ref[:] = (jnp.dot(a_ref[:], wa_ref[:], preferred_element_type=jnp.float32)
                + jnp.dot(b_ref[:], wb_ref[:], preferred_element_type=jnp.float32))


def _mid_body(d, p_ref, b_ref, w_ref, o_ref):
    h = jax.nn.relu(p_ref[0] + p_ref[1] + b_ref[:])[:, :d]
    o_ref[:] = jnp.dot(h, w_ref[:], preferred_element_type=jnp.float32)


def _final_body(d, p_ref, b3_ref, l1w_ref, l1b_ref, l2w_ref, l2b_ref,
                l3w_ref, l3b_ref, o_ref):
    h = jax.nn.relu(p_ref[0] + p_ref[1] + b3_ref[:])[:, :d]
    h = jax.nn.relu(jnp.dot(h, l1w_ref[:].T, preferred_element_type=jnp.float32) + l1b_ref[:])
    h = jax.nn.relu(jnp.dot(h, l2w_ref[:].T, preferred_element_type=jnp.float32) + l2b_ref[:])
    z = jnp.sum(h * l3w_ref[:], axis=1, keepdims=True) + l3b_ref[0]
    o_ref[:] = jax.nn.sigmoid(z)


def _tc_call(body, out_shape, *args):
    return pl.pallas_call(
        body,
        out_shape=jax.ShapeDtypeStruct(out_shape, jnp.float32),
    )(*args)


def _vals_t_body(x_ref, o_ref):
    o_ref[:] = x_ref[:].T


def _vals_transpose(vals):
    blk = 32000
    return pl.pallas_call(
        _vals_t_body,
        grid=(E_EDGES // blk,),
        in_specs=[pl.BlockSpec((blk, N_CH), lambda w: (w, 0))],
        out_specs=pl.BlockSpec((N_CH, blk), lambda w: (0, w)),
        out_shape=jax.ShapeDtypeStruct((N_CH, E_EDGES), jnp.float32),
    )(vals)


def _sc_edge_kernel(dp):
    R = N_CH * dp
    nv = dp // 16

    def body(y_hbm, src_hbm, dst_hbm, vals_hbm, out_hbm,
             src_v, dst_v, vals_v, rows_v, msgs_v, zbuf_v, acc_sh,
             g0, g1, g2, g3, g4, v0, v1, v2, v3, v4, s0, s1, s2, s3, s4):
        gsems = (g0, g1, g2, g3, g4)
        vsems = (v0, v1, v2, v3, v4)
        ssems = (s0, s1, s2, s3, s4)
        cid = lax.axis_index("c")
        sid = lax.axis_index("s")
        wid = cid * NS + sid
        ebase = wid * EPT

        def zrow(r, carry):
            for k in range(nv):
                zbuf_v[r, pl.ds(16 * k, 16)] = jnp.zeros((16,), jnp.float32)
            return carry
        lax.fori_loop(0, NPT // 5, zrow, 0)
        for j in range(5):
            pltpu.sync_copy(zbuf_v,
                            acc_sh.at[pl.ds(sid * NPT + j * (NPT // 5), NPT // 5)])

        pltpu.sync_copy(src_hbm.at[pl.ds(ebase, EPT)], src_v)
        pltpu.sync_copy(dst_hbm.at[wid], dst_v)
        plsc.subcore_barrier()

        def start_rows(i, b):
            pltpu.async_copy(y_hbm.at[src_v.at[pl.ds(i * B, B)]],
                             rows_v.at[b], gsems[b])

        def wait_rows(i, b):
            pltpu.make_async_copy(y_hbm.at[src_v.at[pl.ds(i * B, B)]],
                                  rows_v.at[b], gsems[b]).wait()

        def start_dstvals(i, b):
            pltpu.async_copy(vals_hbm.at[:, pl.ds(ebase + i * B, B)],
                             vals_v.at[b], vsems[b])

        def wait_dstvals(i, b):
            pltpu.make_async_copy(vals_hbm.at[:, pl.ds(ebase + i * B, B)],
                                  vals_v.at[b], vsems[b]).wait()

        def start_scatter(i, b):
            pltpu.async_copy(msgs_v.at[b], acc_sh.at[dst_v.at[i]], ssems[b],
                             add=True)

        def wait_scatter(i, b):
            pltpu.make_async_copy(msgs_v.at[b], acc_sh.at[dst_v.at[i]],
                                  ssems[b]).wait()

        def compute(b):
            def block16(t, c2):
                vcs = [vals_v[b, c, pl.ds(t * 16, 16)] for c in range(N_CH)]
                for le in range(16):
                    e = t * 16 + le
                    accs = [None] * nv
                    for c in range(N_CH):
                        s = vcs[c].at[jnp.full((16,), le, jnp.int32)].get(
                            mode="promise_in_bounds")
                        for k in range(nv):
                            w = s * rows_v[b, e, pl.ds(c * dp + 16 * k, 16)]
                            accs[k] = w if accs[k] is None else accs[k] + w
                    for k in range(nv):
                        msgs_v[b, e, pl.ds(16 * k, 16)] = accs[k]
                return c2
            lax.fori_loop(0, B // 16, block16, 0)

        start_dstvals(0, 0)
        for i in range(NBUF - 1):
            start_rows(i, i)

        def quint(q, carry):
            p0 = q * NBUF
            for b in range(NBUF):
                p = p0 + b

                @pl.when(p >= NBUF - 1)
                def _():
                    wait_scatter(p - (NBUF - 1), (b + 1) % NBUF)

                @pl.when(p + 1 < NB)
                def _():
                    start_dstvals(p + 1, (b + 1) % NBUF)

                @pl.when(p + NBUF - 1 < NB)
                def _():
                    start_rows(p + NBUF - 1, (b + NBUF - 1) % NBUF)
                wait_rows(p, b)
                wait_dstvals(p, b)
                compute(b)
                start_scatter(p, b)
            return carry
        lax.fori_loop(0, NB // NBUF, quint, 0)
        for j in range(NB - NBUF + 1, NB):
            wait_scatter(j, j % NBUF)

        plsc.subcore_barrier()

        @pl.when(sid == 0)
        def _():
            pltpu.sync_copy(acc_sh, out_hbm.at[cid])

    mesh = plsc.VectorSubcoreMesh(core_axis_name="c", subcore_axis_name="s",
                                  num_cores=NC, num_subcores=NS)
    return pl.kernel(
        body,
        out_type=jax.ShapeDtypeStruct((NC, N_NODES, dp), jnp.float32),
        mesh=mesh,
        compiler_params=pltpu.CompilerParams(use_tc_tiling_on_sc=False),
        scratch_types=[
            pltpu.VMEM((EPT,), jnp.int32),
            pltpu.VMEM((NB, B), jnp.int32),
            pltpu.VMEM((NBUF, N_CH, B), jnp.float32),
            pltpu.VMEM((NBUF, B, R), jnp.float32),
            pltpu.VMEM((NBUF, B, dp), jnp.float32),
            pltpu.VMEM((NPT // 5, dp), jnp.float32),
            pltpu.VMEM_SHARED((N_NODES, dp), jnp.float32),
        ] + [pltpu.SemaphoreType.DMA] * 15,
    )


def _stack_pad(W, dp):
    d_in, d = W.shape[1], W.shape[2]
    Wt = jnp.transpose(W, (1, 0, 2))
    Wt = jnp.pad(Wt, ((0, 0), (0, 0), (0, dp - d)))
    return Wt.reshape(d_in, N_CH * dp)


def kernel(one_hot, features, gemme_features, a_res_indices, a_res_values,
           W1, b1, W2, b2, W3, b3, l1w, l1b, l2w, l2b, l3w, l3b):
    src = a_res_indices[0].astype(jnp.int32)
    dst = a_res_indices[1].astype(jnp.int32).reshape(NW, NB, B)
    vals_t = _vals_transpose(a_res_values)

    d1, d2, d3 = W1.shape[2], W2.shape[2], W3.shape[2]
    dp1, dp2, dp3 = 32, 16, 16
    W1s = _stack_pad(W1, dp1)
    W2s = _stack_pad(W2, dp2)
    W3s = _stack_pad(W3, dp3)
    b1p = jnp.pad(b1, (0, dp1 - d1))
    b2p = jnp.pad(b2, (0, dp2 - d2))
    b3p = jnp.pad(b3, (0, dp3 - d3))

    edge1 = _sc_edge_kernel(dp1)
    edge23 = _sc_edge_kernel(dp2)

    r = one_hot.shape[1]
    Y1 = _tc_call(_proj2_body, (N_NODES, N_CH * dp1),
                  one_hot, features, W1s[:r], W1s[r:])
    p1 = edge1(Y1, src, dst, vals_t)

    Y2 = _tc_call(functools.partial(_mid_body, d1), (N_NODES, N_CH * dp2),
                  p1, b1p, W2s)
    p2 = edge23(Y2, src, dst, vals_t)

    Y3 = _tc_call(functools.partial(_mid_body, d2), (N_NODES, N_CH * dp3),
                  p2, b2p, W3s)
    p3 = edge23(Y3, src, dst, vals_t)

    return _tc_call(functools.partial(_final_body, d3), (N_NODES, 1),
                    p3, b3p, l1w, l1b, l2w, l2b, l3w, l3b)
